# baseline (speedup 1.0000x reference)
import jax
import jax.numpy as jnp
from jax.experimental import pallas as pl
from jax.experimental.pallas import tpu as pltpu

_NS = 8


def _body(x_ref, tri_ref, out_ref):
    xb = x_ref[0]
    C, H, W = xb.shape
    colmax = jnp.max(xb, axis=1)
    gm = jnp.max(colmax, axis=1, keepdims=True)
    ties = (xb >= gm[:, :, None]).astype(jnp.float32)
    ones = jnp.ones((C, 1, W), jnp.float32)
    rp = jax.lax.dot_general(
        ones, ties, (((2,), (2,)), ((0,), (0,))),
        preferred_element_type=jnp.float32)
    row = jnp.sum(rp[:, 0, :], axis=0)[None, :]
    cs = jnp.sum(jnp.sum(xb, axis=1), axis=1)

    Hh = jax.lax.dot_general(row, tri_ref[...], (((1,), (0,)), ((), ())),
                             preferred_element_type=jnp.float32)
    lane = jax.lax.broadcasted_iota(jnp.int32, (1, H), 1).astype(jnp.float32)
    inrange = (lane >= 1.0) & (lane <= float(H - 2))
    Hnext = jnp.concatenate([Hh[:, 1:], jnp.zeros((1, 1), jnp.float32)], axis=1)
    BIG = jnp.float32(1e9)
    hk_prev = jnp.zeros((1, 1), jnp.float32)
    valid = jnp.ones((1, 1), jnp.bool_)
    hks = [jnp.zeros((1, 1), jnp.float32)]
    for k in range(1, _NS):
        thr = float(int(k * C / _NS))
        cond = inrange & (Hh <= thr) & (Hnext > thr)
        a = jnp.min(jnp.where(cond, lane, BIG), axis=1, keepdims=True)
        b = jnp.max(jnp.where(cond, lane, -BIG), axis=1, keepdims=True)
        jk = jnp.maximum(a, hk_prev + 1.0)
        valid = valid & (jk <= b)
        hks.append(jnp.where(valid, jk, 0.0))
        hk_prev = jnp.where(valid, jk, hk_prev)
    hks.append(jnp.full((1, 1), jnp.float32(H)))
    hks = jnp.concatenate(hks, axis=1)
    widths = hks[:, 1:] - hks[:, :-1]
    F = cs * jnp.float32(1.0 / W)
    out_ref[0, 0] = F[None, :] / widths[0, :, None]


def _abp(x):
    B, C, H, W = x.shape
    tri = (jax.lax.broadcasted_iota(jnp.int32, (H, H), 0)
           < jax.lax.broadcasted_iota(jnp.int32, (H, H), 1)).astype(jnp.float32)
    out = pl.pallas_call(
        _body,
        grid=(B,),
        in_specs=[
            pl.BlockSpec((1, C, H, W), lambda b: (b, 0, 0, 0)),
            pl.BlockSpec((H, H), lambda b: (0, 0)),
        ],
        out_specs=pl.BlockSpec((1, 1, _NS, C), lambda b: (b, 0, 0, 0)),
        out_shape=jax.ShapeDtypeStruct((B, 1, _NS, C), jnp.float32),
        compiler_params=pltpu.CompilerParams(
            dimension_semantics=("arbitrary",)),
    )(x, tri)
    return out.reshape(B, _NS * C)


def kernel(x):
    return _abp(x)


# speedup vs baseline: 1.0221x; 1.0221x over previous
import jax
import jax.numpy as jnp
from jax.experimental import pallas as pl
from jax.experimental.pallas import tpu as pltpu

_NS = 8


def _body(x_ref, tri_ref, out_ref, row_s, cs_s):
    b = pl.program_id(0)
    B = pl.num_programs(0)
    xb = x_ref[0]
    C, H, W = xb.shape
    colmax = jnp.max(xb, axis=1)
    gm = jnp.max(colmax, axis=1, keepdims=True)
    ties = (xb >= gm[:, :, None]).astype(jnp.float32)
    ones = jnp.ones((C, 1, W), jnp.float32)
    rp = jax.lax.dot_general(
        ones, ties, (((2,), (2,)), ((0,), (0,))),
        preferred_element_type=jnp.float32)
    row_s[b, 0, :] = jnp.sum(rp[:, 0, :], axis=0)
    cs_s[b, 0, :] = jnp.sum(jnp.sum(xb, axis=1), axis=1)

    @pl.when(b == B - 1)
    def _():
        row = row_s[:, 0, :]
        Hh = jax.lax.dot_general(row, tri_ref[...], (((1,), (0,)), ((), ())),
                                 preferred_element_type=jnp.float32)
        lane = jax.lax.broadcasted_iota(jnp.int32, (B, H), 1).astype(jnp.float32)
        inrange = (lane >= 1.0) & (lane <= float(H - 2))
        Hnext = jnp.concatenate(
            [Hh[:, 1:], jnp.zeros((B, 1), jnp.float32)], axis=1)
        BIG = jnp.float32(1e9)
        hk_prev = jnp.zeros((B, 1), jnp.float32)
        valid = jnp.ones((B, 1), jnp.bool_)
        hks = [jnp.zeros((B, 1), jnp.float32)]
        for k in range(1, _NS):
            thr = float(int(k * C / _NS))
            cond = inrange & (Hh <= thr) & (Hnext > thr)
            a = jnp.min(jnp.where(cond, lane, BIG), axis=1, keepdims=True)
            bmax = jnp.max(jnp.where(cond, lane, -BIG), axis=1, keepdims=True)
            jk = jnp.maximum(a, hk_prev + 1.0)
            valid = valid & (jk <= bmax)
            hks.append(jnp.where(valid, jk, 0.0))
            hk_prev = jnp.where(valid, jk, hk_prev)
        hks.append(jnp.full((B, 1), jnp.float32(H)))
        hks = jnp.concatenate(hks, axis=1)
        widths = hks[:, 1:] - hks[:, :-1]
        F = cs_s[:, 0, :] * jnp.float32(1.0 / W)
        out_ref[...] = F[:, None, :] / widths[:, :, None]


def _abp(x):
    B, C, H, W = x.shape
    tri = (jax.lax.broadcasted_iota(jnp.int32, (H, H), 0)
           < jax.lax.broadcasted_iota(jnp.int32, (H, H), 1)).astype(jnp.float32)
    out = pl.pallas_call(
        _body,
        grid=(B,),
        in_specs=[
            pl.BlockSpec((1, C, H, W), lambda b: (b, 0, 0, 0)),
            pl.BlockSpec((H, H), lambda b: (0, 0)),
        ],
        out_specs=pl.BlockSpec((B, _NS, C), lambda b: (0, 0, 0)),
        out_shape=jax.ShapeDtypeStruct((B, _NS, C), jnp.float32),
        scratch_shapes=[
            pltpu.VMEM((B, 1, H), jnp.float32),
            pltpu.VMEM((B, 1, C), jnp.float32),
        ],
        compiler_params=pltpu.CompilerParams(
            dimension_semantics=("arbitrary",)),
    )(x, tri)
    return out.reshape(B, _NS * C)


def kernel(x):
    return _abp(x)


# speedup vs baseline: 1.0393x; 1.0168x over previous
import jax
import jax.numpy as jnp
from jax.experimental import pallas as pl
from jax.experimental.pallas import tpu as pltpu

_NS = 8


def _body(x_ref, out_ref, row_s, cs_s):
    b = pl.program_id(0)
    B = pl.num_programs(0)
    xb = x_ref[0]
    C, H, W = xb.shape
    colmax = jnp.max(xb, axis=1)
    gm = jnp.max(colmax, axis=1, keepdims=True)
    ties = (xb >= gm[:, :, None]).astype(jnp.float32)
    ones = jnp.ones((C, 1, W), jnp.float32)
    rp = jax.lax.dot_general(
        ones, ties, (((2,), (2,)), ((0,), (0,))),
        preferred_element_type=jnp.float32)
    row_s[b, 0, :] = jnp.sum(rp[:, 0, :], axis=0)
    cs_s[b, 0, :] = jnp.sum(jnp.sum(xb, axis=1), axis=1)

    @pl.when(b == B - 1)
    def _():
        row = row_s[:, 0, :]
        tri = (jax.lax.broadcasted_iota(jnp.int32, (H, H), 0)
               < jax.lax.broadcasted_iota(jnp.int32, (H, H), 1)
               ).astype(jnp.float32)
        Hh = jax.lax.dot_general(row, tri, (((1,), (0,)), ((), ())),
                                 preferred_element_type=jnp.float32)
        lane = jax.lax.broadcasted_iota(jnp.int32, (B, H), 1).astype(jnp.float32)
        inrange = (lane >= 1.0) & (lane <= float(H - 2))
        Hnext = jnp.concatenate(
            [Hh[:, 1:], jnp.zeros((B, 1), jnp.float32)], axis=1)
        BIG = jnp.float32(1e9)
        hk_prev = jnp.zeros((B, 1), jnp.float32)
        valid = jnp.ones((B, 1), jnp.bool_)
        hks = [jnp.zeros((B, 1), jnp.float32)]
        for k in range(1, _NS):
            thr = float(int(k * C / _NS))
            cond = inrange & (Hh <= thr) & (Hnext > thr)
            a = jnp.min(jnp.where(cond, lane, BIG), axis=1, keepdims=True)
            bmax = jnp.max(jnp.where(cond, lane, -BIG), axis=1, keepdims=True)
            jk = jnp.maximum(a, hk_prev + 1.0)
            valid = valid & (jk <= bmax)
            hks.append(jnp.where(valid, jk, 0.0))
            hk_prev = jnp.where(valid, jk, hk_prev)
        hks.append(jnp.full((B, 1), jnp.float32(H)))
        hks = jnp.concatenate(hks, axis=1)
        widths = hks[:, 1:] - hks[:, :-1]
        F = cs_s[:, 0, :] * jnp.float32(1.0 / W)
        out_ref[...] = F[:, None, :] / widths[:, :, None]


def _abp(x):
    B, C, H, W = x.shape
    out = pl.pallas_call(
        _body,
        grid=(B,),
        in_specs=[pl.BlockSpec((1, C, H, W), lambda b: (b, 0, 0, 0))],
        out_specs=pl.BlockSpec((B, _NS, C), lambda b: (0, 0, 0)),
        out_shape=jax.ShapeDtypeStruct((B, _NS, C), jnp.float32),
        scratch_shapes=[
            pltpu.VMEM((B, 1, H), jnp.float32),
            pltpu.VMEM((B, 1, C), jnp.float32),
        ],
        compiler_params=pltpu.CompilerParams(
            dimension_semantics=("arbitrary",)),
    )(x)
    return out.reshape(B, _NS * C)


def kernel(x):
    return _abp(x)
